# baseline (speedup 1.0000x reference)
import functools

import jax
import jax.numpy as jnp
from jax import lax
from jax.experimental import pallas as pl
from jax.experimental.pallas import tpu as pltpu
from jax.experimental.pallas import tpu_sc as plsc

HID = 1024
EXPERTS = 8
TOPK = 2
ROWS = 8192
TM = 256
S = ROWS + EXPERTS * TM
NW = 32


def _routing(ids, wt):
    oh = (ids[:, None] == jnp.arange(EXPERTS, dtype=ids.dtype)[None, :]).astype(jnp.int32)
    csum = jnp.cumsum(oh, axis=0)
    counts = csum[-1]
    rank = jnp.take_along_axis(csum, ids[:, None], axis=1)[:, 0] - 1
    padded = ((counts + TM - 1) // TM) * TM
    ends = jnp.cumsum(padded)
    offsets = ends - padded
    pos = offsets[ids] + rank
    inv = jnp.zeros((S,), jnp.int32).at[pos].set(jnp.arange(ROWS, dtype=jnp.int32))
    wsort = jnp.zeros((S,), jnp.float32).at[pos].set(wt)
    tile_starts = jnp.arange(S // TM, dtype=jnp.int32) * TM
    tile_expert = jnp.minimum(
        jnp.searchsorted(ends, tile_starts, side="right"), EXPERTS - 1
    ).astype(jnp.int32)
    return pos, inv, wsort, tile_expert


_G_CH = 64
_G_NBUF = 3


def _sc_gather(table, idx):
    B = idx.shape[0]
    D = table.shape[1]
    b_per_w = B // NW
    n_ch = b_per_w // _G_CH
    mesh = plsc.VectorSubcoreMesh(core_axis_name="c", subcore_axis_name="s")

    @functools.partial(
        pl.kernel,
        mesh=mesh,
        out_type=jax.ShapeDtypeStruct((B, D), table.dtype),
        scratch_types=[
            pltpu.VMEM((b_per_w,), jnp.int32),
            [pltpu.VMEM((_G_CH, D), table.dtype) for _ in range(_G_NBUF)],
            pltpu.SemaphoreType.DMA,
            pltpu.SemaphoreType.DMA,
        ],
    )
    def k(table_hbm, idx_hbm, out_hbm, idx_v, bufs, sem_g, sem_w):
        wid = lax.axis_index("s") * 2 + lax.axis_index("c")
        base = pl.multiple_of(wid * b_per_w, _G_CH)
        pltpu.sync_copy(idx_hbm.at[pl.ds(base, b_per_w)], idx_v)

        def start_gather(c):
            return pltpu.async_copy(
                table_hbm.at[idx_v.at[pl.ds(c * _G_CH, _G_CH)]],
                bufs[c % _G_NBUF], sem_g)

        gathers = {}
        for c in range(min(_G_NBUF, n_ch)):
            gathers[c] = start_gather(c)
        writes = {}
        for c in range(n_ch):
            gathers[c].wait()
            writes[c] = pltpu.async_copy(
                bufs[c % _G_NBUF],
                out_hbm.at[pl.ds(base + c * _G_CH, _G_CH)], sem_w)
            nxt = c + _G_NBUF
            if nxt < n_ch:
                writes[c].wait()
                del writes[c]
                gathers[nxt] = start_gather(nxt)
        for c in list(writes):
            writes[c].wait()

    return k(table, idx)


_C_CH = 16


def _sc_combine(y, p0, p1):
    T = p0.shape[0]
    D = y.shape[1]
    t_per_w = T // NW
    n_ch = t_per_w // _C_CH
    mesh = plsc.VectorSubcoreMesh(core_axis_name="c", subcore_axis_name="s")

    @functools.partial(
        pl.kernel,
        mesh=mesh,
        out_type=jax.ShapeDtypeStruct((T, D), jnp.float32),
        scratch_types=[
            pltpu.VMEM((t_per_w,), jnp.int32),
            pltpu.VMEM((t_per_w,), jnp.int32),
            [pltpu.VMEM((_C_CH, D), jnp.float32) for _ in range(2)],
            [pltpu.VMEM((_C_CH, D), jnp.float32) for _ in range(2)],
            pltpu.SemaphoreType.DMA,
            pltpu.SemaphoreType.DMA,
        ],
    )
    def k(y_hbm, p0_hbm, p1_hbm, out_hbm, i0_v, i1_v, a_bufs, b_bufs, sem_g, sem_w):
        wid = lax.axis_index("s") * 2 + lax.axis_index("c")
        base = pl.multiple_of(wid * t_per_w, _C_CH)
        pltpu.sync_copy(p0_hbm.at[pl.ds(base, t_per_w)], i0_v)
        pltpu.sync_copy(p1_hbm.at[pl.ds(base, t_per_w)], i1_v)

        def start_gathers(c):
            sl = pl.ds(c * _C_CH, _C_CH)
            return (
                pltpu.async_copy(y_hbm.at[i0_v.at[sl]], a_bufs[c % 2], sem_g),
                pltpu.async_copy(y_hbm.at[i1_v.at[sl]], b_bufs[c % 2], sem_g),
            )

        gathers = {0: start_gathers(0)}
        if n_ch > 1:
            gathers[1] = start_gathers(1)
        writes = {}
        for c in range(n_ch):
            ga, gb = gathers[c]
            ga.wait()
            gb.wait()
            a_v, b_v = a_bufs[c % 2], b_bufs[c % 2]

            def add_row(r, _):
                for j in range(D // 16):
                    sl = pl.ds(j * 16, 16)
                    a_v[r, sl] = a_v[r, sl] + b_v[r, sl]
                return ()

            lax.fori_loop(0, _C_CH, add_row, ())
            writes[c] = pltpu.async_copy(
                a_v, out_hbm.at[pl.ds(base + c * _C_CH, _C_CH)], sem_w)
            if c + 2 < n_ch:
                writes[c].wait()
                del writes[c]
                gathers[c + 2] = start_gathers(c + 2)
        for c in list(writes):
            writes[c].wait()

    return k(y, p0, p1)


def _gemm_body(te_ref, x_ref, w_ref, wv_ref, y_ref):
    y = jnp.dot(x_ref[...], w_ref[0], preferred_element_type=jnp.float32)
    y_ref[...] = y * wv_ref[...]


def _grouped_gemm(x_sorted, w_bf, wsort, tile_expert):
    grid_spec = pltpu.PrefetchScalarGridSpec(
        num_scalar_prefetch=1,
        grid=(S // TM,),
        in_specs=[
            pl.BlockSpec((TM, HID), lambda i, te: (i, 0)),
            pl.BlockSpec((1, HID, HID), lambda i, te: (te[i], 0, 0)),
            pl.BlockSpec((TM, 1), lambda i, te: (i, 0)),
        ],
        out_specs=pl.BlockSpec((TM, HID), lambda i, te: (i, 0)),
    )
    return pl.pallas_call(
        _gemm_body,
        grid_spec=grid_spec,
        out_shape=jax.ShapeDtypeStruct((S, HID), jnp.float32),
    )(tile_expert, x_sorted, w_bf, wsort[:, None])


def kernel(intermediate_states, w, full_topk_ids, full_topk_weight):
    num_tokens = ROWS // TOPK
    ids = full_topk_ids[:num_tokens].reshape(-1)
    wt = full_topk_weight[:num_tokens].reshape(-1)

    pos, inv, wsort, tile_expert = _routing(ids, wt)

    x_bf = intermediate_states.astype(jnp.bfloat16)
    x_i32 = jax.lax.bitcast_convert_type(x_bf.reshape(ROWS, HID // 2, 2), jnp.int32)
    xs_i32 = _sc_gather(x_i32, inv)
    x_sorted = jax.lax.bitcast_convert_type(xs_i32, jnp.bfloat16).reshape(S, HID)
    w_bf = w.astype(jnp.bfloat16)
    y_sorted = _grouped_gemm(x_sorted, w_bf, wsort, tile_expert)

    p0 = pos[0::2]
    p1 = pos[1::2]
    return _sc_combine(y_sorted, p0, p1)


# speedup vs baseline: 2.0038x; 2.0038x over previous
import functools

import jax
import jax.numpy as jnp
from jax import lax
from jax.experimental import pallas as pl
from jax.experimental.pallas import tpu as pltpu
from jax.experimental.pallas import tpu_sc as plsc

HID = 1024
EXPERTS = 8
TOPK = 2
ROWS = 8192
TM = 256
S = ROWS + EXPERTS * TM
NW = 32


def _routing(ids, wt):
    oh = (ids[:, None] == jnp.arange(EXPERTS, dtype=ids.dtype)[None, :]).astype(jnp.int32)
    csum = jnp.cumsum(oh, axis=0)
    counts = csum[-1]
    rank = jnp.take_along_axis(csum, ids[:, None], axis=1)[:, 0] - 1
    padded = ((counts + TM - 1) // TM) * TM
    ends = jnp.cumsum(padded)
    offsets = ends - padded
    pos = offsets[ids] + rank
    inv = jnp.zeros((S,), jnp.int32).at[pos].set(jnp.arange(ROWS, dtype=jnp.int32))
    wsort = jnp.zeros((S,), jnp.float32).at[pos].set(wt)
    tile_starts = jnp.arange(S // TM, dtype=jnp.int32) * TM
    tile_expert = jnp.minimum(
        jnp.searchsorted(ends, tile_starts, side="right"), EXPERTS - 1
    ).astype(jnp.int32)
    return pos, inv, wsort, tile_expert


_G_CH = 32
_G_NBUF = 3


def _sc_gather(table, idx):
    B = idx.shape[0]
    D = table.shape[1]
    b_per_w = B // NW
    n_ch = b_per_w // _G_CH
    mesh = plsc.VectorSubcoreMesh(core_axis_name="c", subcore_axis_name="s")

    @functools.partial(
        pl.kernel,
        mesh=mesh,
        out_type=jax.ShapeDtypeStruct((B, D), table.dtype),
        scratch_types=[
            pltpu.VMEM((b_per_w,), jnp.int32),
            [pltpu.VMEM((_G_CH, D), table.dtype) for _ in range(_G_NBUF)],
            pltpu.SemaphoreType.DMA,
            pltpu.SemaphoreType.DMA,
        ],
    )
    def k(table_hbm, idx_hbm, out_hbm, idx_v, bufs, sem_g, sem_w):
        wid = lax.axis_index("s") * 2 + lax.axis_index("c")
        base = pl.multiple_of(wid * b_per_w, _G_CH)
        pltpu.sync_copy(idx_hbm.at[pl.ds(base, b_per_w)], idx_v)

        def start_gather(c):
            return pltpu.async_copy(
                table_hbm.at[idx_v.at[pl.ds(c * _G_CH, _G_CH)]],
                bufs[c % _G_NBUF], sem_g)

        gathers = {}
        for c in range(min(_G_NBUF, n_ch)):
            gathers[c] = start_gather(c)
        writes = {}
        for c in range(n_ch):
            gathers[c].wait()
            writes[c] = pltpu.async_copy(
                bufs[c % _G_NBUF],
                out_hbm.at[pl.ds(base + c * _G_CH, _G_CH)], sem_w)
            nxt = c + _G_NBUF
            if nxt < n_ch:
                writes[c].wait()
                del writes[c]
                gathers[nxt] = start_gather(nxt)
        for c in list(writes):
            writes[c].wait()

    return k(table, idx)


_C_CH = 16


def _sc_combine(y, pos):
    T = pos.shape[0] // 2
    D = y.shape[1]
    t_per_w = T // NW
    n_ch = t_per_w // _C_CH
    mesh = plsc.VectorSubcoreMesh(core_axis_name="c", subcore_axis_name="s")

    @functools.partial(
        pl.kernel,
        mesh=mesh,
        out_type=jax.ShapeDtypeStruct((T, D), jnp.float32),
        scratch_types=[
            pltpu.VMEM((2 * t_per_w,), jnp.int32),
            [pltpu.VMEM((2 * _C_CH, D), jnp.float32) for _ in range(2)],
            pltpu.SemaphoreType.DMA,
            pltpu.SemaphoreType.DMA,
        ],
    )
    def k(y_hbm, pos_hbm, out_hbm, idx_v, bufs, sem_g, sem_w):
        wid = lax.axis_index("s") * 2 + lax.axis_index("c")
        base = pl.multiple_of(wid * t_per_w, _C_CH)
        pltpu.sync_copy(pos_hbm.at[pl.ds(2 * base, 2 * t_per_w)], idx_v)

        def start_gather(c):
            return pltpu.async_copy(
                y_hbm.at[idx_v.at[pl.ds(c * 2 * _C_CH, 2 * _C_CH)]],
                bufs[c % 2], sem_g)

        gathers = {0: start_gather(0)}
        if n_ch > 1:
            gathers[1] = start_gather(1)
        writes = {}
        for c in range(n_ch):
            gathers[c].wait()
            buf = bufs[c % 2]

            def add_row(r, _):
                for j in range(D // 16):
                    sl = pl.ds(j * 16, 16)
                    buf[r, sl] = buf[2 * r, sl] + buf[2 * r + 1, sl]
                return ()

            lax.fori_loop(0, _C_CH, add_row, ())
            writes[c] = pltpu.async_copy(
                buf.at[pl.ds(0, _C_CH)],
                out_hbm.at[pl.ds(base + c * _C_CH, _C_CH)], sem_w)
            if c + 2 < n_ch:
                writes[c].wait()
                del writes[c]
                gathers[c + 2] = start_gather(c + 2)
        for c in list(writes):
            writes[c].wait()

    return k(y, pos)


def _gemm_body(te_ref, x_ref, w_ref, wv_ref, y_ref):
    x = x_ref[...].astype(jnp.bfloat16)
    y = jnp.dot(x, w_ref[0], preferred_element_type=jnp.float32)
    y_ref[...] = y * wv_ref[...]


def _grouped_gemm(x_sorted, w_bf, wsort, tile_expert):
    grid_spec = pltpu.PrefetchScalarGridSpec(
        num_scalar_prefetch=1,
        grid=(S // TM,),
        in_specs=[
            pl.BlockSpec((TM, HID), lambda i, te: (i, 0)),
            pl.BlockSpec((1, HID, HID), lambda i, te: (te[i], 0, 0)),
            pl.BlockSpec((TM, 1), lambda i, te: (i, 0)),
        ],
        out_specs=pl.BlockSpec((TM, HID), lambda i, te: (i, 0)),
    )
    return pl.pallas_call(
        _gemm_body,
        grid_spec=grid_spec,
        out_shape=jax.ShapeDtypeStruct((S, HID), jnp.float32),
    )(tile_expert, x_sorted, w_bf, wsort[:, None])


def kernel(intermediate_states, w, full_topk_ids, full_topk_weight):
    num_tokens = ROWS // TOPK
    ids = full_topk_ids[:num_tokens].reshape(-1)
    wt = full_topk_weight[:num_tokens].reshape(-1)

    pos, inv, wsort, tile_expert = _routing(ids, wt)

    x_sorted = _sc_gather(intermediate_states, inv)
    w_bf = w.astype(jnp.bfloat16)
    y_sorted = _grouped_gemm(x_sorted, w_bf, wsort, tile_expert)

    return _sc_combine(y_sorted, pos)
